# baseline (speedup 1.0000x reference)
import functools

import jax
import jax.numpy as jnp
from jax import lax
from jax.experimental import pallas as pl
from jax.experimental.pallas import tpu as pltpu
from jax.experimental.pallas import tpu_sc as plsc

_NC = 2
_NS = 16
_NW = _NC * _NS
_CHUNK = 128
_NBUF = 2


def _segment_sum_sc(x, zeros, src1, dst3, n, d):
    ch = dst3.shape[1]
    epw = ch * _CHUNK
    rows_per_tile = (n // _NS) // 8 * 8
    tail_row0 = rows_per_tile * _NS
    tail_rows = n - tail_row0

    mesh = plsc.VectorSubcoreMesh(core_axis_name="c", subcore_axis_name="s")

    @functools.partial(
        pl.kernel,
        out_type=jax.ShapeDtypeStruct((_NC, n, d), jnp.float32),
        mesh=mesh,
        scratch_types=[
            pltpu.VMEM((epw,), jnp.int32),
            pltpu.VMEM((ch // 2, _CHUNK), jnp.int32),
            pltpu.VMEM((_NBUF, _CHUNK, d), jnp.float32),
            pltpu.VMEM_SHARED((n, d), jnp.float32),
            [pltpu.SemaphoreType.DMA] * _NBUF,
            [pltpu.SemaphoreType.DMA] * _NBUF,
        ],
    )
    def seg_sum(x_hbm, z_hbm, src_hbm, dst_hbm, out_hbm,
                src_v, dst_v, rows_v, acc_sh, sems, ssems):
        rows = [rows_v.at[b] for b in range(_NBUF)]
        cid = lax.axis_index("c")
        sid = lax.axis_index("s")
        wid = sid * _NC + cid
        row0 = sid * rows_per_tile
        pltpu.sync_copy(z_hbm.at[pl.ds(row0, rows_per_tile)],
                        acc_sh.at[pl.ds(row0, rows_per_tile)])

        @pl.when(sid == _NS - 1)
        def _zero_tail():
            pltpu.sync_copy(z_hbm.at[pl.ds(tail_row0, tail_rows)],
                            acc_sh.at[pl.ds(tail_row0, tail_rows)])

        hch = ch // 2
        pltpu.sync_copy(src_hbm.at[pl.ds(wid * epw, epw)], src_v)
        pltpu.sync_copy(dst_hbm.at[wid, pl.ds(0, hch)], dst_v)
        plsc.subcore_barrier()

        def src_idx(j):
            return src_v.at[pl.ds(pl.multiple_of(j * _CHUNK, 8), _CHUNK)]

        for b in range(_NBUF - 1):
            pltpu.async_copy(x_hbm.at[src_idx(b)], rows[b], sems[b])

        def body(k, carry):
            j0 = k * _NBUF
            for b in range(_NBUF):
                j = j0 + b
                nxt = j + _NBUF - 1
                nb = (b + _NBUF - 1) % _NBUF

                @pl.when(nxt < ch)
                def _start_next():
                    @pl.when(nxt >= _NBUF)
                    def _drain_prev_scatter():
                        pltpu.make_async_copy(
                            rows[nb], acc_sh.at[dst_v.at[0]], ssems[nb]).wait()

                    pltpu.async_copy(x_hbm.at[src_idx(nxt)], rows[nb],
                                     sems[nb])

                pltpu.make_async_copy(x_hbm.at[src_idx(j)], rows[b],
                                      sems[b]).wait()

                @pl.when(j == hch)
                def _reload_dst():
                    pltpu.sync_copy(dst_hbm.at[wid, pl.ds(hch, hch)], dst_v)

                jj = lax.select(j >= hch, j - hch, j)
                pltpu.async_copy(rows[b], acc_sh.at[dst_v.at[jj]], ssems[b],
                                 add=True)
            return carry

        lax.fori_loop(0, ch // _NBUF, body, 0)
        for b in range(_NBUF):
            pltpu.make_async_copy(rows[b], acc_sh.at[dst_v.at[0]],
                                  ssems[b]).wait()
        plsc.subcore_barrier()
        pltpu.sync_copy(acc_sh.at[pl.ds(row0, rows_per_tile)],
                        out_hbm.at[cid, pl.ds(row0, rows_per_tile)])

        @pl.when(sid == _NS - 1)
        def _out_tail():
            pltpu.sync_copy(acc_sh.at[pl.ds(tail_row0, tail_rows)],
                            out_hbm.at[cid, pl.ds(tail_row0, tail_rows)])

    return seg_sum(x, zeros, src1, dst3)


def _mlp_pre_tc(x, w1t, b1, eps, n, d):
    blk = 1000
    grid = (n // blk,)

    def body(eps_ref, x_ref, w1_ref, b1_ref, o_ref):
        scale = 1.0 + eps_ref[0]
        h = jnp.dot(x_ref[...], w1_ref[...], preferred_element_type=jnp.float32)
        o_ref[...] = scale * h + b1_ref[...]

    return pl.pallas_call(
        body,
        grid=grid,
        in_specs=[
            pl.BlockSpec(memory_space=pltpu.SMEM),
            pl.BlockSpec((blk, d), lambda i: (i, 0)),
            pl.BlockSpec((d, d), lambda i: (0, 0)),
            pl.BlockSpec((1, d), lambda i: (0, 0)),
        ],
        out_specs=pl.BlockSpec((blk, d), lambda i: (i, 0)),
        out_shape=jax.ShapeDtypeStruct((n, d), jnp.float32),
    )(eps, x, w1t, b1)


def _mlp_post_tc(p, a, x, w1t, w2t, b2, pad_total, n, d):
    blk = 1000
    grid = (n // blk,)

    def body(p_ref, a_ref, x_ref, w1_ref, w2_ref, b2_ref, o_ref):
        s = p_ref[0] + p_ref[1]
        if pad_total:
            row = (lax.broadcasted_iota(jnp.int32, (blk, d), 0)
                   + pl.program_id(0) * blk)
            s = s - jnp.where(row < pad_total, 1.0, 0.0) * x_ref[...]
        h = jnp.dot(s, w1_ref[...], preferred_element_type=jnp.float32)
        h = jnp.maximum(h + a_ref[...], 0.0)
        o = jnp.dot(h, w2_ref[...], preferred_element_type=jnp.float32)
        o_ref[...] = o + b2_ref[...]

    return pl.pallas_call(
        body,
        grid=grid,
        in_specs=[
            pl.BlockSpec((2, blk, d), lambda i: (0, i, 0)),
            pl.BlockSpec((blk, d), lambda i: (i, 0)),
            pl.BlockSpec((blk, d), lambda i: (i, 0)),
            pl.BlockSpec((d, d), lambda i: (0, 0)),
            pl.BlockSpec((d, d), lambda i: (0, 0)),
            pl.BlockSpec((1, d), lambda i: (0, 0)),
        ],
        out_specs=pl.BlockSpec((blk, d), lambda i: (i, 0)),
        out_shape=jax.ShapeDtypeStruct((n, d), jnp.float32),
    )(p, a, x, w1t, w2t, b2)


def kernel(x, edge_index, W1, b1, W2, b2, eps):
    n, d = x.shape
    e = edge_index.shape[1]
    src = edge_index[0].astype(jnp.int32)
    dst = edge_index[1].astype(jnp.int32)
    epw = e // _NW
    ch = 2 * (-(-epw // (2 * _CHUNK)))
    pad = ch * _CHUNK - epw
    pad_total = pad * _NW
    assert pad_total <= n, "pad self-loops must map to distinct nodes"
    pad_idx = (jnp.arange(_NW, dtype=jnp.int32)[:, None] * pad
               + jnp.arange(pad, dtype=jnp.int32)[None, :])
    src1 = jnp.concatenate([src.reshape(_NW, epw), pad_idx], 1).reshape(-1)
    dst3 = jnp.concatenate([dst.reshape(_NW, epw), pad_idx], 1).reshape(
        _NW, ch, _CHUNK)
    zeros = jnp.zeros((n, d), jnp.float32)
    w1t = W1.T
    p = _segment_sum_sc(x, zeros, src1, dst3, n, d)
    a = _mlp_pre_tc(x, w1t, b1.reshape(1, d), eps, n, d)
    return _mlp_post_tc(p, a, x, w1t, W2.T, b2.reshape(1, d), pad_total, n, d)


# speedup vs baseline: 1.0120x; 1.0120x over previous
import functools

import jax
import jax.numpy as jnp
from jax import lax
from jax.experimental import pallas as pl
from jax.experimental.pallas import tpu as pltpu
from jax.experimental.pallas import tpu_sc as plsc

_NC = 2
_NS = 16
_NW = _NC * _NS
_CHUNK = 128
_NBUF = 2


def _segment_sum_sc(x, zeros, src1, dst3, n, d):
    ch = dst3.shape[1]
    epw = ch * _CHUNK
    rows_per_tile = (n // _NS) // 8 * 8
    tail_row0 = rows_per_tile * _NS
    tail_rows = n - tail_row0

    mesh = plsc.VectorSubcoreMesh(core_axis_name="c", subcore_axis_name="s")

    @functools.partial(
        pl.kernel,
        out_type=jax.ShapeDtypeStruct((_NC, n, d), jnp.float32),
        mesh=mesh,
        scratch_types=[
            pltpu.VMEM((epw,), jnp.int32),
            pltpu.VMEM((ch // 2, _CHUNK), jnp.int32),
            pltpu.VMEM((_NBUF, _CHUNK, d), jnp.float32),
            pltpu.VMEM_SHARED((n, d), jnp.float32),
            [pltpu.SemaphoreType.DMA] * _NBUF,
            [pltpu.SemaphoreType.DMA] * _NBUF,
        ],
    )
    def seg_sum(x_hbm, z_hbm, src_hbm, dst_hbm, out_hbm,
                src_v, dst_v, rows_v, acc_sh, sems, ssems):
        rows = [rows_v.at[b] for b in range(_NBUF)]
        cid = lax.axis_index("c")
        sid = lax.axis_index("s")
        wid = sid * _NC + cid
        row0 = sid * rows_per_tile
        pltpu.sync_copy(z_hbm.at[pl.ds(row0, rows_per_tile)],
                        acc_sh.at[pl.ds(row0, rows_per_tile)])

        @pl.when(sid == _NS - 1)
        def _zero_tail():
            pltpu.sync_copy(z_hbm.at[pl.ds(tail_row0, tail_rows)],
                            acc_sh.at[pl.ds(tail_row0, tail_rows)])

        hch = ch // 2
        pltpu.sync_copy(src_hbm.at[pl.ds(wid * epw, epw)], src_v)
        pltpu.sync_copy(dst_hbm.at[wid, pl.ds(0, hch)], dst_v)
        plsc.subcore_barrier()

        def src_idx(j):
            return src_v.at[pl.ds(pl.multiple_of(j * _CHUNK, 8), _CHUNK)]

        for b in range(_NBUF - 1):
            pltpu.async_copy(x_hbm.at[src_idx(b)], rows[b], sems[b])

        def body(k, carry):
            j0 = k * _NBUF
            for b in range(_NBUF):
                j = j0 + b
                nxt = j + _NBUF - 1
                nb = (b + _NBUF - 1) % _NBUF

                @pl.when(nxt < ch)
                def _start_next():
                    @pl.when(nxt >= _NBUF)
                    def _drain_prev_scatter():
                        pltpu.make_async_copy(
                            rows[nb], acc_sh.at[dst_v.at[0]], ssems[nb]).wait()

                    pltpu.async_copy(x_hbm.at[src_idx(nxt)], rows[nb],
                                     sems[nb])

                pltpu.make_async_copy(x_hbm.at[src_idx(j)], rows[b],
                                      sems[b]).wait()

                @pl.when(j == hch)
                def _reload_dst():
                    pltpu.sync_copy(dst_hbm.at[wid, pl.ds(hch, hch)], dst_v)

                jj = lax.select(j >= hch, j - hch, j)
                pltpu.async_copy(rows[b], acc_sh.at[dst_v.at[jj]], ssems[b],
                                 add=True)
            return carry

        lax.fori_loop(0, ch // _NBUF, body, 0)
        for b in range(_NBUF):
            pltpu.make_async_copy(rows[b], acc_sh.at[dst_v.at[0]],
                                  ssems[b]).wait()
        plsc.subcore_barrier()
        pltpu.sync_copy(acc_sh.at[pl.ds(row0, rows_per_tile)],
                        out_hbm.at[cid, pl.ds(row0, rows_per_tile)])

        @pl.when(sid == _NS - 1)
        def _out_tail():
            pltpu.sync_copy(acc_sh.at[pl.ds(tail_row0, tail_rows)],
                            out_hbm.at[cid, pl.ds(tail_row0, tail_rows)])

    return seg_sum(x, zeros, src1, dst3)


def _mlp_pre_tc(x, w1t, b1, eps, pad_total, n, d):
    blk = 1000
    grid = (n // blk,)

    def body(eps_ref, x_ref, w1_ref, b1_ref, o_ref):
        scale = 1.0 + eps_ref[0]
        if pad_total:
            row = (lax.broadcasted_iota(jnp.int32, (blk, d), 0)
                   + pl.program_id(0) * blk)
            scale = scale - jnp.where(row < pad_total, 1.0, 0.0)
        h = jnp.dot(x_ref[...], w1_ref[...], preferred_element_type=jnp.float32)
        o_ref[...] = scale * h + b1_ref[...]

    return pl.pallas_call(
        body,
        grid=grid,
        in_specs=[
            pl.BlockSpec(memory_space=pltpu.SMEM),
            pl.BlockSpec((blk, d), lambda i: (i, 0)),
            pl.BlockSpec((d, d), lambda i: (0, 0)),
            pl.BlockSpec((1, d), lambda i: (0, 0)),
        ],
        out_specs=pl.BlockSpec((blk, d), lambda i: (i, 0)),
        out_shape=jax.ShapeDtypeStruct((n, d), jnp.float32),
    )(eps, x, w1t, b1)


def _mlp_post_tc(p, a, w1t, w2t, b2, n, d):
    blk = 1000
    grid = (n // blk,)

    def body(p_ref, a_ref, w1_ref, w2_ref, b2_ref, o_ref):
        s = p_ref[0] + p_ref[1]
        h = jnp.dot(s, w1_ref[...], preferred_element_type=jnp.float32)
        h = jnp.maximum(h + a_ref[...], 0.0)
        o = jnp.dot(h, w2_ref[...], preferred_element_type=jnp.float32)
        o_ref[...] = o + b2_ref[...]

    return pl.pallas_call(
        body,
        grid=grid,
        in_specs=[
            pl.BlockSpec((2, blk, d), lambda i: (0, i, 0)),
            pl.BlockSpec((blk, d), lambda i: (i, 0)),
            pl.BlockSpec((d, d), lambda i: (0, 0)),
            pl.BlockSpec((d, d), lambda i: (0, 0)),
            pl.BlockSpec((1, d), lambda i: (0, 0)),
        ],
        out_specs=pl.BlockSpec((blk, d), lambda i: (i, 0)),
        out_shape=jax.ShapeDtypeStruct((n, d), jnp.float32),
    )(p, a, w1t, w2t, b2)


def kernel(x, edge_index, W1, b1, W2, b2, eps):
    n, d = x.shape
    e = edge_index.shape[1]
    src = edge_index[0].astype(jnp.int32)
    dst = edge_index[1].astype(jnp.int32)
    epw = e // _NW
    ch = 2 * (-(-epw // (2 * _CHUNK)))
    pad = ch * _CHUNK - epw
    pad_total = pad * _NW
    assert pad_total <= n, "pad self-loops must map to distinct nodes"
    pad_idx = (jnp.arange(_NW, dtype=jnp.int32)[:, None] * pad
               + jnp.arange(pad, dtype=jnp.int32)[None, :])
    src1 = jnp.concatenate([src.reshape(_NW, epw), pad_idx], 1).reshape(-1)
    dst3 = jnp.concatenate([dst.reshape(_NW, epw), pad_idx], 1).reshape(
        _NW, ch, _CHUNK)
    zeros = jnp.zeros((n, d), jnp.float32)
    w1t = W1.T
    p = _segment_sum_sc(x, zeros, src1, dst3, n, d)
    a = _mlp_pre_tc(x, w1t, b1.reshape(1, d), eps, pad_total, n, d)
    return _mlp_post_tc(p, a, w1t, W2.T, b2.reshape(1, d), n, d)


# speedup vs baseline: 1.0139x; 1.0019x over previous
import functools

import jax
import jax.numpy as jnp
from jax import lax
from jax.experimental import pallas as pl
from jax.experimental.pallas import tpu as pltpu
from jax.experimental.pallas import tpu_sc as plsc

_NC = 2
_NS = 16
_NW = _NC * _NS
_CHUNK = 128
_NBUF = 2


def _segment_sum_sc(x, zeros, src1, dst3, n, d):
    ch = dst3.shape[1]
    epw = ch * _CHUNK
    rows_per_tile = (n // _NS) // 8 * 8
    tail_row0 = rows_per_tile * _NS
    tail_rows = n - tail_row0

    mesh = plsc.VectorSubcoreMesh(core_axis_name="c", subcore_axis_name="s")

    @functools.partial(
        pl.kernel,
        out_type=jax.ShapeDtypeStruct((_NC, n, d), jnp.float32),
        mesh=mesh,
        scratch_types=[
            pltpu.VMEM((epw,), jnp.int32),
            pltpu.VMEM((ch // 2, _CHUNK), jnp.int32),
            pltpu.VMEM((_NBUF, _CHUNK, d), jnp.float32),
            pltpu.VMEM_SHARED((n, d), jnp.float32),
            [pltpu.SemaphoreType.DMA] * _NBUF,
            [pltpu.SemaphoreType.DMA] * _NBUF,
        ],
    )
    def seg_sum(x_hbm, z_hbm, src_hbm, dst_hbm, out_hbm,
                src_v, dst_v, rows_v, acc_sh, sems, ssems):
        rows = [rows_v.at[b] for b in range(_NBUF)]
        cid = lax.axis_index("c")
        sid = lax.axis_index("s")
        wid = sid * _NC + cid
        row0 = sid * rows_per_tile
        pltpu.sync_copy(z_hbm.at[pl.ds(row0, rows_per_tile)],
                        acc_sh.at[pl.ds(row0, rows_per_tile)])

        @pl.when(sid == _NS - 1)
        def _zero_tail():
            pltpu.sync_copy(z_hbm.at[pl.ds(tail_row0, tail_rows)],
                            acc_sh.at[pl.ds(tail_row0, tail_rows)])

        hch = ch // 2
        pltpu.sync_copy(src_hbm.at[pl.ds(wid * epw, epw)], src_v)
        pltpu.sync_copy(dst_hbm.at[wid, pl.ds(0, hch)], dst_v)
        plsc.subcore_barrier()

        def src_idx(j):
            return src_v.at[pl.ds(pl.multiple_of(j * _CHUNK, 8), _CHUNK)]

        for b in range(_NBUF - 1):
            pltpu.async_copy(x_hbm.at[src_idx(b)], rows[b], sems[b])

        def body(k, carry):
            j0 = k * _NBUF
            for b in range(_NBUF):
                j = j0 + b
                nxt = j + _NBUF - 1
                nb = (b + _NBUF - 1) % _NBUF

                @pl.when(nxt < ch)
                def _start_next():
                    @pl.when(nxt >= _NBUF)
                    def _drain_prev_scatter():
                        pltpu.make_async_copy(
                            rows[nb], acc_sh.at[dst_v.at[0]], ssems[nb]).wait()

                    pltpu.async_copy(x_hbm.at[src_idx(nxt)], rows[nb],
                                     sems[nb])

                pltpu.make_async_copy(x_hbm.at[src_idx(j)], rows[b],
                                      sems[b]).wait()

                @pl.when(j == hch)
                def _reload_dst():
                    pltpu.sync_copy(dst_hbm.at[wid, pl.ds(hch, hch)], dst_v)

                jj = lax.select(j >= hch, j - hch, j)
                pltpu.async_copy(rows[b], acc_sh.at[dst_v.at[jj]], ssems[b],
                                 add=True)
            return carry

        lax.fori_loop(0, ch // _NBUF, body, 0)
        for b in range(_NBUF):
            pltpu.make_async_copy(rows[b], acc_sh.at[dst_v.at[0]],
                                  ssems[b]).wait()
        plsc.subcore_barrier()
        pltpu.sync_copy(acc_sh.at[pl.ds(row0, rows_per_tile)],
                        out_hbm.at[cid, pl.ds(row0, rows_per_tile)])

        @pl.when(sid == _NS - 1)
        def _out_tail():
            pltpu.sync_copy(acc_sh.at[pl.ds(tail_row0, tail_rows)],
                            out_hbm.at[cid, pl.ds(tail_row0, tail_rows)])

    return seg_sum(x, zeros, src1, dst3)


def _mlp_pre_tc(x, w1t, b1, eps, pad_total, n, d):
    blk = 1000
    grid = (n // blk,)

    def body(eps_ref, x_ref, w1_ref, b1_ref, o_ref):
        scale = 1.0 + eps_ref[0]
        if pad_total:
            row = (lax.broadcasted_iota(jnp.int32, (blk, d), 0)
                   + pl.program_id(0) * blk)
            scale = scale - jnp.where(row < pad_total, 1.0, 0.0)
        h = jnp.dot(x_ref[...], w1_ref[...], preferred_element_type=jnp.float32)
        o_ref[...] = scale * h + b1_ref[...]

    return pl.pallas_call(
        body,
        grid=grid,
        in_specs=[
            pl.BlockSpec(memory_space=pltpu.SMEM),
            pl.BlockSpec((blk, d), lambda i: (i, 0)),
            pl.BlockSpec((d, d), lambda i: (0, 0)),
            pl.BlockSpec((1, d), lambda i: (0, 0)),
        ],
        out_specs=pl.BlockSpec((blk, d), lambda i: (i, 0)),
        out_shape=jax.ShapeDtypeStruct((n, d), jnp.float32),
    )(eps, x, w1t, b1)


def _mlp_fused_tc(p, x, w1t, b1, w2t, b2, eps, pad_total, n, d):
    blk = 1000
    grid = (n // blk,)

    def body(eps_ref, p_ref, x_ref, w1_ref, b1_ref, w2_ref, b2_ref, o_ref):
        scale = 1.0 + eps_ref[0]
        if pad_total:
            row = (lax.broadcasted_iota(jnp.int32, (blk, d), 0)
                   + pl.program_id(0) * blk)
            scale = scale - jnp.where(row < pad_total, 1.0, 0.0)
        y = p_ref[0] + p_ref[1] + scale * x_ref[...]
        h = jnp.dot(y, w1_ref[...], preferred_element_type=jnp.float32)
        h = jnp.maximum(h + b1_ref[...], 0.0)
        o = jnp.dot(h, w2_ref[...], preferred_element_type=jnp.float32)
        o_ref[...] = o + b2_ref[...]

    return pl.pallas_call(
        body,
        grid=grid,
        in_specs=[
            pl.BlockSpec(memory_space=pltpu.SMEM),
            pl.BlockSpec((2, blk, d), lambda i: (0, i, 0)),
            pl.BlockSpec((blk, d), lambda i: (i, 0)),
            pl.BlockSpec((d, d), lambda i: (0, 0)),
            pl.BlockSpec((1, d), lambda i: (0, 0)),
            pl.BlockSpec((d, d), lambda i: (0, 0)),
            pl.BlockSpec((1, d), lambda i: (0, 0)),
        ],
        out_specs=pl.BlockSpec((blk, d), lambda i: (i, 0)),
        out_shape=jax.ShapeDtypeStruct((n, d), jnp.float32),
    )(eps, p, x, w1t, b1, w2t, b2)


def kernel(x, edge_index, W1, b1, W2, b2, eps):
    n, d = x.shape
    e = edge_index.shape[1]
    src = edge_index[0].astype(jnp.int32)
    dst = edge_index[1].astype(jnp.int32)
    epw = e // _NW
    ch = 2 * (-(-epw // (2 * _CHUNK)))
    pad = ch * _CHUNK - epw
    pad_total = pad * _NW
    assert pad_total <= n, "pad self-loops must map to distinct nodes"
    pad_idx = (jnp.arange(_NW, dtype=jnp.int32)[:, None] * pad
               + jnp.arange(pad, dtype=jnp.int32)[None, :])
    src1 = jnp.concatenate([src.reshape(_NW, epw), pad_idx], 1).reshape(-1)
    dst3 = jnp.concatenate([dst.reshape(_NW, epw), pad_idx], 1).reshape(
        _NW, ch, _CHUNK)
    zeros = jnp.zeros((n, d), jnp.float32)
    w1t = W1.T
    p = _segment_sum_sc(x, zeros, src1, dst3, n, d)
    return _mlp_fused_tc(p, x, w1t, b1.reshape(1, d), W2.T, b2.reshape(1, d),
                         eps, pad_total, n, d)


# speedup vs baseline: 1.0368x; 1.0226x over previous
import functools

import jax
import jax.numpy as jnp
from jax import lax
from jax.experimental import pallas as pl
from jax.experimental.pallas import tpu as pltpu
from jax.experimental.pallas import tpu_sc as plsc

_NC = 2
_NS = 16
_NW = _NC * _NS
_CHUNK = 128
_NBUF = 2


def _segment_sum_sc(x, zeros, src1, dst3, n, d):
    ch = dst3.shape[1]
    epw = ch * _CHUNK
    rows_per_tile = (n // _NS) // 8 * 8
    tail_row0 = rows_per_tile * _NS
    tail_rows = n - tail_row0

    mesh = plsc.VectorSubcoreMesh(core_axis_name="c", subcore_axis_name="s")

    @functools.partial(
        pl.kernel,
        out_type=jax.ShapeDtypeStruct((_NC, n, d), jnp.float32),
        mesh=mesh,
        scratch_types=[
            pltpu.VMEM((epw,), jnp.int32),
            pltpu.VMEM((ch // 2, _CHUNK), jnp.int32),
            pltpu.VMEM((_NBUF, _CHUNK, d), jnp.float32),
            pltpu.VMEM_SHARED((n, d), jnp.float32),
            [pltpu.SemaphoreType.DMA] * _NBUF,
            [pltpu.SemaphoreType.DMA] * _NBUF,
        ],
    )
    def seg_sum(x_hbm, z_hbm, src_hbm, dst_hbm, out_hbm,
                src_v, dst_v, rows_v, acc_sh, sems, ssems):
        rows = [rows_v.at[b] for b in range(_NBUF)]
        cid = lax.axis_index("c")
        sid = lax.axis_index("s")
        wid = sid * _NC + cid
        row0 = sid * rows_per_tile
        pltpu.sync_copy(z_hbm.at[pl.ds(row0, rows_per_tile)],
                        acc_sh.at[pl.ds(row0, rows_per_tile)])

        @pl.when(sid == _NS - 1)
        def _zero_tail():
            pltpu.sync_copy(z_hbm.at[pl.ds(tail_row0, tail_rows)],
                            acc_sh.at[pl.ds(tail_row0, tail_rows)])

        hch = ch // 2
        pltpu.sync_copy(src_hbm.at[pl.ds(wid * epw, epw)], src_v)
        pltpu.sync_copy(dst_hbm.at[wid, pl.ds(0, hch)], dst_v)
        plsc.subcore_barrier()

        def src_idx(j):
            return src_v.at[pl.ds(pl.multiple_of(j * _CHUNK, 8), _CHUNK)]

        for b in range(_NBUF - 1):
            pltpu.async_copy(x_hbm.at[src_idx(b)], rows[b], sems[b])

        def body(k, carry):
            j0 = k * _NBUF
            for b in range(_NBUF):
                j = j0 + b
                nxt = j + _NBUF - 1
                nb = (b + _NBUF - 1) % _NBUF

                @pl.when(nxt < ch)
                def _start_next():
                    @pl.when(nxt >= _NBUF)
                    def _drain_prev_scatter():
                        pltpu.make_async_copy(
                            rows[nb], acc_sh.at[dst_v.at[0]], ssems[nb]).wait()

                    pltpu.async_copy(x_hbm.at[src_idx(nxt)], rows[nb],
                                     sems[nb])

                pltpu.make_async_copy(x_hbm.at[src_idx(j)], rows[b],
                                      sems[b]).wait()

                @pl.when(j == hch)
                def _reload_dst():
                    pltpu.sync_copy(dst_hbm.at[wid, pl.ds(hch, hch)], dst_v)

                jj = lax.select(j >= hch, j - hch, j)
                pltpu.async_copy(rows[b], acc_sh.at[dst_v.at[jj]], ssems[b],
                                 add=True)
            return carry

        lax.fori_loop(0, ch // _NBUF, body, 0)
        for b in range(_NBUF):
            pltpu.make_async_copy(rows[b], acc_sh.at[dst_v.at[0]],
                                  ssems[b]).wait()
        plsc.subcore_barrier()
        pltpu.sync_copy(acc_sh.at[pl.ds(row0, rows_per_tile)],
                        out_hbm.at[cid, pl.ds(row0, rows_per_tile)])

        @pl.when(sid == _NS - 1)
        def _out_tail():
            pltpu.sync_copy(acc_sh.at[pl.ds(tail_row0, tail_rows)],
                            out_hbm.at[cid, pl.ds(tail_row0, tail_rows)])

    return seg_sum(x, zeros, src1, dst3)


def _mlp_fused_tc(p, x, w1t, b1, w2t, b2, eps, pad_total, n, d):
    blk = 2000
    grid = (n // blk,)

    def body(eps_ref, p_ref, x_ref, w1_ref, b1_ref, w2_ref, b2_ref, o_ref):
        scale = 1.0 + eps_ref[0]
        if pad_total:
            row = (lax.broadcasted_iota(jnp.int32, (blk, d), 0)
                   + pl.program_id(0) * blk)
            scale = scale - jnp.where(row < pad_total, 1.0, 0.0)
        y = p_ref[0] + p_ref[1] + scale * x_ref[...]
        h = jnp.dot(y, w1_ref[...], preferred_element_type=jnp.float32)
        h = jnp.maximum(h + b1_ref[...], 0.0)
        o = jnp.dot(h, w2_ref[...], preferred_element_type=jnp.float32)
        o_ref[...] = o + b2_ref[...]

    return pl.pallas_call(
        body,
        grid=grid,
        in_specs=[
            pl.BlockSpec(memory_space=pltpu.SMEM),
            pl.BlockSpec((2, blk, d), lambda i: (0, i, 0)),
            pl.BlockSpec((blk, d), lambda i: (i, 0)),
            pl.BlockSpec((d, d), lambda i: (0, 0)),
            pl.BlockSpec((1, d), lambda i: (0, 0)),
            pl.BlockSpec((d, d), lambda i: (0, 0)),
            pl.BlockSpec((1, d), lambda i: (0, 0)),
        ],
        out_specs=pl.BlockSpec((blk, d), lambda i: (i, 0)),
        out_shape=jax.ShapeDtypeStruct((n, d), jnp.float32),
    )(eps, p, x, w1t, b1, w2t, b2)


def kernel(x, edge_index, W1, b1, W2, b2, eps):
    n, d = x.shape
    e = edge_index.shape[1]
    src = edge_index[0].astype(jnp.int32)
    dst = edge_index[1].astype(jnp.int32)
    epw = e // _NW
    ch = 2 * (-(-epw // (2 * _CHUNK)))
    pad = ch * _CHUNK - epw
    pad_total = pad * _NW
    assert pad_total <= n, "pad self-loops must map to distinct nodes"
    pad_idx = (jnp.arange(_NW, dtype=jnp.int32)[:, None] * pad
               + jnp.arange(pad, dtype=jnp.int32)[None, :])
    src1 = jnp.concatenate([src.reshape(_NW, epw), pad_idx], 1).reshape(-1)
    dst3 = jnp.concatenate([dst.reshape(_NW, epw), pad_idx], 1).reshape(
        _NW, ch, _CHUNK)
    zeros = jnp.zeros((n, d), jnp.float32)
    w1t = W1.T
    p = _segment_sum_sc(x, zeros, src1, dst3, n, d)
    return _mlp_fused_tc(p, x, w1t, b1.reshape(1, d), W2.T, b2.reshape(1, d),
                         eps, pad_total, n, d)


# speedup vs baseline: 1.0400x; 1.0030x over previous
import functools

import jax
import jax.numpy as jnp
from jax import lax
from jax.experimental import pallas as pl
from jax.experimental.pallas import tpu as pltpu
from jax.experimental.pallas import tpu_sc as plsc

_NC = 2
_NS = 16
_NW = _NC * _NS
_CHUNK = 128
_NBUF = 2


def _segment_sum_sc(x, zeros, src1, dst3, n, d):
    ch = dst3.shape[1]
    epw = ch * _CHUNK
    rows_per_tile = (n // _NS) // 8 * 8
    tail_row0 = rows_per_tile * _NS
    tail_rows = n - tail_row0

    mesh = plsc.VectorSubcoreMesh(core_axis_name="c", subcore_axis_name="s")

    @functools.partial(
        pl.kernel,
        out_type=jax.ShapeDtypeStruct((_NC, n, d), jnp.float32),
        mesh=mesh,
        scratch_types=[
            pltpu.VMEM((epw,), jnp.int32),
            pltpu.VMEM((ch // 2, _CHUNK), jnp.int32),
            pltpu.VMEM((_NBUF, _CHUNK, d), jnp.float32),
            pltpu.VMEM_SHARED((n, d), jnp.float32),
            [pltpu.SemaphoreType.DMA] * _NBUF,
            [pltpu.SemaphoreType.DMA] * _NBUF,
        ],
    )
    def seg_sum(x_hbm, z_hbm, src_hbm, dst_hbm, out_hbm,
                src_v, dst_v, rows_v, acc_sh, sems, ssems):
        rows = [rows_v.at[b] for b in range(_NBUF)]
        cid = lax.axis_index("c")
        sid = lax.axis_index("s")
        wid = sid * _NC + cid
        row0 = sid * rows_per_tile
        pltpu.sync_copy(z_hbm.at[pl.ds(row0, rows_per_tile)],
                        acc_sh.at[pl.ds(row0, rows_per_tile)])

        @pl.when(sid == _NS - 1)
        def _zero_tail():
            pltpu.sync_copy(z_hbm.at[pl.ds(tail_row0, tail_rows)],
                            acc_sh.at[pl.ds(tail_row0, tail_rows)])

        hch = ch // 2
        pltpu.sync_copy(src_hbm.at[pl.ds(wid * epw, epw)], src_v)
        pltpu.sync_copy(dst_hbm.at[wid, pl.ds(0, hch)], dst_v)
        plsc.subcore_barrier()

        def src_idx(j):
            return src_v.at[pl.ds(pl.multiple_of(j * _CHUNK, 8), _CHUNK)]

        for b in range(_NBUF - 1):
            pltpu.async_copy(x_hbm.at[src_idx(b)], rows[b], sems[b])

        def body(k, carry):
            j0 = k * _NBUF
            for b in range(_NBUF):
                j = j0 + b
                nxt = j + _NBUF - 1
                nb = (b + _NBUF - 1) % _NBUF

                @pl.when(nxt < ch)
                def _start_next():
                    @pl.when(nxt >= _NBUF)
                    def _drain_prev_scatter():
                        pltpu.make_async_copy(
                            rows[nb], acc_sh.at[dst_v.at[0]], ssems[nb]).wait()

                    pltpu.async_copy(x_hbm.at[src_idx(nxt)], rows[nb],
                                     sems[nb])

                pltpu.make_async_copy(x_hbm.at[src_idx(j)], rows[b],
                                      sems[b]).wait()

                @pl.when(j == hch)
                def _reload_dst():
                    pltpu.sync_copy(dst_hbm.at[wid, pl.ds(hch, hch)], dst_v)

                jj = lax.select(j >= hch, j - hch, j)
                pltpu.async_copy(rows[b], acc_sh.at[dst_v.at[jj]], ssems[b],
                                 add=True)
            return carry

        lax.fori_loop(0, ch // _NBUF, body, 0)
        for b in range(_NBUF):
            pltpu.make_async_copy(rows[b], acc_sh.at[dst_v.at[0]],
                                  ssems[b]).wait()
        plsc.subcore_barrier()
        pltpu.sync_copy(acc_sh.at[pl.ds(row0, rows_per_tile)],
                        out_hbm.at[cid, pl.ds(row0, rows_per_tile)])

        @pl.when(sid == _NS - 1)
        def _out_tail():
            pltpu.sync_copy(acc_sh.at[pl.ds(tail_row0, tail_rows)],
                            out_hbm.at[cid, pl.ds(tail_row0, tail_rows)])

    return seg_sum(x, zeros, src1, dst3)


def _mlp_fused_tc(p, x, w1t, b1, w2t, b2, eps, pad_total, n, d):
    blk = 5000
    grid = (n // blk,)

    def body(eps_ref, p_ref, x_ref, w1_ref, b1_ref, w2_ref, b2_ref, o_ref):
        scale = 1.0 + eps_ref[0]
        if pad_total:
            row = (lax.broadcasted_iota(jnp.int32, (blk, d), 0)
                   + pl.program_id(0) * blk)
            scale = scale - jnp.where(row < pad_total, 1.0, 0.0)
        y = p_ref[0] + p_ref[1] + scale * x_ref[...]
        h = jnp.dot(y, w1_ref[...], preferred_element_type=jnp.float32)
        h = jnp.maximum(h + b1_ref[...], 0.0)
        o = jnp.dot(h, w2_ref[...], preferred_element_type=jnp.float32)
        o_ref[...] = o + b2_ref[...]

    return pl.pallas_call(
        body,
        grid=grid,
        in_specs=[
            pl.BlockSpec(memory_space=pltpu.SMEM),
            pl.BlockSpec((2, blk, d), lambda i: (0, i, 0)),
            pl.BlockSpec((blk, d), lambda i: (i, 0)),
            pl.BlockSpec((d, d), lambda i: (0, 0)),
            pl.BlockSpec((1, d), lambda i: (0, 0)),
            pl.BlockSpec((d, d), lambda i: (0, 0)),
            pl.BlockSpec((1, d), lambda i: (0, 0)),
        ],
        out_specs=pl.BlockSpec((blk, d), lambda i: (i, 0)),
        out_shape=jax.ShapeDtypeStruct((n, d), jnp.float32),
    )(eps, p, x, w1t, b1, w2t, b2)


def kernel(x, edge_index, W1, b1, W2, b2, eps):
    n, d = x.shape
    e = edge_index.shape[1]
    src = edge_index[0].astype(jnp.int32)
    dst = edge_index[1].astype(jnp.int32)
    epw = e // _NW
    ch = 2 * (-(-epw // (2 * _CHUNK)))
    pad = ch * _CHUNK - epw
    pad_total = pad * _NW
    assert pad_total <= n, "pad self-loops must map to distinct nodes"
    pad_idx = (jnp.arange(_NW, dtype=jnp.int32)[:, None] * pad
               + jnp.arange(pad, dtype=jnp.int32)[None, :])
    src1 = jnp.concatenate([src.reshape(_NW, epw), pad_idx], 1).reshape(-1)
    dst3 = jnp.concatenate([dst.reshape(_NW, epw), pad_idx], 1).reshape(
        _NW, ch, _CHUNK)
    zeros = jnp.zeros((n, d), jnp.float32)
    w1t = W1.T
    p = _segment_sum_sc(x, zeros, src1, dst3, n, d)
    return _mlp_fused_tc(p, x, w1t, b1.reshape(1, d), W2.T, b2.reshape(1, d),
                         eps, pad_total, n, d)
